# baseline (speedup 1.0000x reference)
import functools

import jax
import jax.numpy as jnp
from jax import lax
from jax.experimental import pallas as pl
from jax.experimental.pallas import tpu as pltpu
from jax.experimental.pallas import tpu_sc as plsc

BSZ = 4
NTILE = 4
NTOK = 1601
DIM = 1280
NSLAB = BSZ * NTILE
TB = 267


def _pos_body(ar_hbm, emb_hbm, pos_hbm, ar_v, idx_v, rows_v, sem):
  pltpu.sync_copy(ar_hbm, ar_v)

  lanes = lax.iota(jnp.int32, 16)
  four = jnp.full((16,), NTILE, jnp.int32)
  one16 = jnp.full((16,), 1, jnp.int32)
  t_vec = lax.rem(lanes, four)
  h_vec = ar_v[pl.ds(0, 16)]
  w_vec = ar_v[pl.ds(16, 16)]
  w_safe = jnp.maximum(w_vec, one16)
  row = lax.div(t_vec, w_safe)
  col = lax.rem(t_vec, w_safe)
  m = jnp.minimum(jnp.maximum(h_vec * w_vec - t_vec, one16 - one16), one16)
  emb_idx = m * (row * four + col) + (one16 - m) * jnp.full(
      (16,), NSLAB, jnp.int32)
  idx_v[...] = emb_idx

  gcp = pltpu.make_async_copy(emb_hbm.at[idx_v], rows_v, sem)
  gcp.start()
  gcp.wait()
  cps = [pltpu.make_async_copy(rows_v.at[pl.ds(b * NTILE, NTILE)],
                               pos_hbm.at[b], sem) for b in range(BSZ)]
  for cp in cps:
    cp.start()
  for cp in cps:
    cp.wait()


def _sc_pos_table(ar32x, emb2):
  mesh = plsc.VectorSubcoreMesh(core_axis_name="c", subcore_axis_name="s",
                                num_cores=1, num_subcores=1)
  run = functools.partial(
      pl.kernel,
      out_type=jax.ShapeDtypeStruct((BSZ, NTILE, DIM), jnp.float32),
      mesh=mesh,
      compiler_params=pltpu.CompilerParams(skip_device_barrier=True),
      scratch_types=[
          pltpu.VMEM((32,), jnp.int32),
          pltpu.VMEM((16,), jnp.int32),
          pltpu.VMEM((NSLAB, DIM), jnp.float32),
          pltpu.SemaphoreType.DMA,
      ],
  )(_pos_body)
  return run(ar32x, emb2)


def _add_body(x_ref, pos_ref, gate_ref, o_ref):
  g = jnp.tanh(gate_ref[0, 0])
  o_ref[...] = x_ref[...] + pos_ref[...][:, None] * g


def _tc_add(xt, pos4, gate2):
  grid = (BSZ, pl.cdiv(NTOK, TB))
  return pl.pallas_call(
      _add_body,
      grid=grid,
      in_specs=[
          pl.BlockSpec((1, TB, NTILE, DIM), lambda b, j: (b, j, 0, 0)),
          pl.BlockSpec((1, NTILE, DIM), lambda b, j: (b, 0, 0)),
          pl.BlockSpec(memory_space=pltpu.SMEM),
      ],
      out_specs=pl.BlockSpec((1, TB, NTILE, DIM), lambda b, j: (b, j, 0, 0)),
      out_shape=jax.ShapeDtypeStruct((BSZ, NTOK, NTILE, DIM), jnp.float32),
  )(xt, pos4, gate2)


@jax.jit
def kernel(x, aspect_ratio, embedding, gate):
  ar32 = aspect_ratio.astype(jnp.int32)
  ar32x = jnp.broadcast_to(ar32.T.reshape(8)[:, None], (8, NTILE)).reshape(32)
  gate2 = gate.astype(jnp.float32).reshape(1, 1)
  emb2 = jnp.concatenate(
      [embedding.reshape(NSLAB, DIM),
       jnp.zeros((1, DIM), jnp.float32)])

  pos = _sc_pos_table(ar32x, emb2)
  xt = x.transpose(0, 2, 1, 3)
  out = _tc_add(xt, pos, gate2)
  return out.transpose(0, 2, 1, 3)


# speedup vs baseline: 1.0130x; 1.0130x over previous
import functools

import jax
import jax.numpy as jnp
from jax import lax
from jax.experimental import pallas as pl
from jax.experimental.pallas import tpu as pltpu
from jax.experimental.pallas import tpu_sc as plsc

BSZ = 4
NTILE = 4
NTOK = 1601
DIM = 1280
NSLAB = BSZ * NTILE
TB = 534


def _pos_body(ar_hbm, emb_hbm, pos_hbm, ar_v, idx_v, rows_v, sem):
  pltpu.sync_copy(ar_hbm, ar_v)

  lanes = lax.iota(jnp.int32, 16)
  four = jnp.full((16,), NTILE, jnp.int32)
  one16 = jnp.full((16,), 1, jnp.int32)
  t_vec = lax.rem(lanes, four)
  h_vec = ar_v[pl.ds(0, 16)]
  w_vec = ar_v[pl.ds(16, 16)]
  w_safe = jnp.maximum(w_vec, one16)
  row = lax.div(t_vec, w_safe)
  col = lax.rem(t_vec, w_safe)
  m = jnp.minimum(jnp.maximum(h_vec * w_vec - t_vec, one16 - one16), one16)
  emb_idx = m * (row * four + col) + (one16 - m) * jnp.full(
      (16,), NSLAB, jnp.int32)
  idx_v[...] = emb_idx

  gcp = pltpu.make_async_copy(emb_hbm.at[idx_v], rows_v, sem)
  gcp.start()
  gcp.wait()
  cps = [pltpu.make_async_copy(rows_v.at[pl.ds(b * NTILE, NTILE)],
                               pos_hbm.at[b], sem) for b in range(BSZ)]
  for cp in cps:
    cp.start()
  for cp in cps:
    cp.wait()


def _sc_pos_table(ar32x, emb2):
  mesh = plsc.VectorSubcoreMesh(core_axis_name="c", subcore_axis_name="s",
                                num_cores=1, num_subcores=1)
  run = functools.partial(
      pl.kernel,
      out_type=jax.ShapeDtypeStruct((BSZ, NTILE, DIM), jnp.float32),
      mesh=mesh,
      compiler_params=pltpu.CompilerParams(skip_device_barrier=True),
      scratch_types=[
          pltpu.VMEM((32,), jnp.int32),
          pltpu.VMEM((16,), jnp.int32),
          pltpu.VMEM((NSLAB, DIM), jnp.float32),
          pltpu.SemaphoreType.DMA,
      ],
  )(_pos_body)
  return run(ar32x, emb2)


def _add_body(x_ref, pos_ref, gate_ref, o_ref):
  g = jnp.tanh(gate_ref[0, 0])
  o_ref[...] = x_ref[...] + pos_ref[...][:, None] * g


def _tc_add(xt, pos4, gate2):
  grid = (BSZ, pl.cdiv(NTOK, TB))
  return pl.pallas_call(
      _add_body,
      grid=grid,
      in_specs=[
          pl.BlockSpec((1, TB, NTILE, DIM), lambda b, j: (b, j, 0, 0)),
          pl.BlockSpec((1, NTILE, DIM), lambda b, j: (b, 0, 0)),
          pl.BlockSpec(memory_space=pltpu.SMEM),
      ],
      out_specs=pl.BlockSpec((1, TB, NTILE, DIM), lambda b, j: (b, j, 0, 0)),
      out_shape=jax.ShapeDtypeStruct((BSZ, NTOK, NTILE, DIM), jnp.float32),
  )(xt, pos4, gate2)


@jax.jit
def kernel(x, aspect_ratio, embedding, gate):
  ar32 = aspect_ratio.astype(jnp.int32)
  ar32x = jnp.broadcast_to(ar32.T.reshape(8)[:, None], (8, NTILE)).reshape(32)
  gate2 = gate.astype(jnp.float32).reshape(1, 1)
  emb2 = jnp.concatenate(
      [embedding.reshape(NSLAB, DIM),
       jnp.zeros((1, DIM), jnp.float32)])

  pos = _sc_pos_table(ar32x, emb2)
  xt = x.transpose(0, 2, 1, 3)
  out = _tc_add(xt, pos, gate2)
  return out.transpose(0, 2, 1, 3)


# speedup vs baseline: 1.0139x; 1.0010x over previous
import functools

import jax
import jax.numpy as jnp
from jax import lax
from jax.experimental import pallas as pl
from jax.experimental.pallas import tpu as pltpu
from jax.experimental.pallas import tpu_sc as plsc

BSZ = 4
NTILE = 4
NTOK = 1601
DIM = 1280
NSLAB = BSZ * NTILE
TB = 534


def _pos_body(ar_hbm, emb_hbm, pos_hbm, ar_v, idx_v, rows_v, sem):
  pltpu.sync_copy(ar_hbm, ar_v)

  lanes = lax.iota(jnp.int32, 16)
  four = jnp.full((16,), NTILE, jnp.int32)
  one16 = jnp.full((16,), 1, jnp.int32)
  t_vec = lax.rem(lanes, four)
  b2_vec = lax.div(lanes, four) * 2
  ar_vec = ar_v[pl.ds(0, 16)]
  h_vec = ar_vec.at[b2_vec].get(mode="promise_in_bounds")
  w_vec = ar_vec.at[b2_vec + one16].get(mode="promise_in_bounds")
  w_safe = jnp.maximum(w_vec, one16)
  row = lax.div(t_vec, w_safe)
  col = lax.rem(t_vec, w_safe)
  m = jnp.minimum(jnp.maximum(h_vec * w_vec - t_vec, one16 - one16), one16)
  emb_idx = m * (row * four + col) + (one16 - m) * jnp.full(
      (16,), NSLAB, jnp.int32)
  idx_v[...] = emb_idx

  gcp = pltpu.make_async_copy(emb_hbm.at[idx_v], rows_v, sem)
  gcp.start()
  gcp.wait()
  cps = [pltpu.make_async_copy(rows_v.at[pl.ds(b * NTILE, NTILE)],
                               pos_hbm.at[b], sem) for b in range(BSZ)]
  for cp in cps:
    cp.start()
  for cp in cps:
    cp.wait()


def _sc_pos_table(ar32x, emb2):
  mesh = plsc.VectorSubcoreMesh(core_axis_name="c", subcore_axis_name="s",
                                num_cores=1, num_subcores=1)
  run = functools.partial(
      pl.kernel,
      out_type=jax.ShapeDtypeStruct((BSZ, NTILE, DIM), jnp.float32),
      mesh=mesh,
      compiler_params=pltpu.CompilerParams(skip_device_barrier=True),
      scratch_types=[
          pltpu.VMEM((16,), jnp.int32),
          pltpu.VMEM((16,), jnp.int32),
          pltpu.VMEM((NSLAB, DIM), jnp.float32),
          pltpu.SemaphoreType.DMA,
      ],
  )(_pos_body)
  return run(ar32x, emb2)


def _add_body(x_ref, pos_ref, gate_ref, o_ref):
  g = jnp.tanh(gate_ref[0, 0])
  o_ref[...] = x_ref[...] + pos_ref[...][:, None] * g


def _tc_add(xt, pos4, gate2):
  grid = (BSZ, pl.cdiv(NTOK, TB))
  return pl.pallas_call(
      _add_body,
      grid=grid,
      in_specs=[
          pl.BlockSpec((1, TB, NTILE, DIM), lambda b, j: (b, j, 0, 0)),
          pl.BlockSpec((1, NTILE, DIM), lambda b, j: (b, 0, 0)),
          pl.BlockSpec(memory_space=pltpu.SMEM),
      ],
      out_specs=pl.BlockSpec((1, TB, NTILE, DIM), lambda b, j: (b, j, 0, 0)),
      out_shape=jax.ShapeDtypeStruct((BSZ, NTOK, NTILE, DIM), jnp.float32),
  )(xt, pos4, gate2)


@jax.jit
def kernel(x, aspect_ratio, embedding, gate):
  ar32x = jnp.zeros((16,), jnp.int32).at[:8].set(
      aspect_ratio.astype(jnp.int32).reshape(8))
  gate2 = gate.astype(jnp.float32).reshape(1, 1)
  emb2 = jnp.concatenate(
      [embedding.reshape(NSLAB, DIM),
       jnp.zeros((1, DIM), jnp.float32)])

  pos = _sc_pos_table(ar32x, emb2)
  xt = x.transpose(0, 2, 1, 3)
  out = _tc_add(xt, pos, gate2)
  return out.transpose(0, 2, 1, 3)
